# Initial kernel scaffold; baseline (speedup 1.0000x reference)
import functools

import jax
import jax.numpy as jnp
from jax import lax
from jax.experimental import pallas as pl
from jax.experimental.pallas import tpu as pltpu
from jax.experimental.pallas import tpu_sc as plsc

NN = 10000
EE = 320000
DD = 128

NWORK = 32
NPW = 320
NPAD = NWORK * NPW
WIN = 2000
GW = 256
CAP = EE + 8 * WIN

_mesh = plsc.VectorSubcoreMesh(core_axis_name="c", subcore_axis_name="s")


def _wid():
    return lax.axis_index("s") * 2 + lax.axis_index("c")


@jax.jit
def _partition(src_all, dst_all):
    @functools.partial(
        pl.kernel,
        mesh=_mesh,
        out_type=[
            jax.ShapeDtypeStruct((NWORK, 16), jnp.int32),
            jax.ShapeDtypeStruct((NWORK, CAP), jnp.int32),
            jax.ShapeDtypeStruct((NWORK, CAP), jnp.int32),
        ],
        scratch_types=[
            pltpu.VMEM((WIN,), jnp.int32),
            pltpu.VMEM((WIN,), jnp.int32),
            pltpu.VMEM((WIN,), jnp.int32),
            pltpu.VMEM((WIN,), jnp.int32),
            pltpu.VMEM((16,), jnp.int32),
        ],
    )
    def part_kernel(src_hbm, dst_hbm, cnt_hbm, srcl_hbm, dstl_hbm,
                    srcw, dstw, stg_s, stg_d, cntv):
        wid = _wid()
        lo = wid * NPW
        hi = lo + NPW

        sent_s = jnp.full((16,), 0, jnp.int32) + lo
        sent_d = jnp.full((16,), NPW, jnp.int32)

        @pl.loop(0, WIN // 16)
        def _(i):
            stg_s[pl.ds(i * 16, 16)] = sent_s
            stg_d[pl.ds(i * 16, 16)] = sent_d

        def win_body(g, total):
            pltpu.sync_copy(src_hbm.at[pl.ds(g * WIN, WIN)], srcw)
            pltpu.sync_copy(dst_hbm.at[pl.ds(g * WIN, WIN)], dstw)

            def chunk(i, cnt):
                d = dstw[pl.ds(i * 16, 16)]
                s = srcw[pl.ds(i * 16, 16)]
                msk = (d >= lo) & (d < hi)
                pos = plsc.cumsum(msk.astype(jnp.int32)) + (cnt - 1)
                plsc.store_scatter(stg_s, [pos], s, msk)
                plsc.store_scatter(stg_d, [pos], d - lo, msk)
                pc = plsc.all_reduce_population_count(msk)
                return cnt + pc[0]

            cnt = lax.fori_loop(0, WIN // 16, chunk, 0)
            pltpu.sync_copy(stg_s, srcl_hbm.at[wid, pl.ds(total, WIN)])
            pltpu.sync_copy(stg_d, dstl_hbm.at[wid, pl.ds(total, WIN)])
            return total + ((cnt + 15) & ~15)

        total = lax.fori_loop(0, EE // WIN, win_body, 0)
        pltpu.sync_copy(stg_s, srcl_hbm.at[wid, pl.ds(total, WIN)])
        pltpu.sync_copy(stg_d, dstl_hbm.at[wid, pl.ds(total, WIN)])
        cntv[...] = jnp.full((16,), 0, jnp.int32) + total
        pltpu.sync_copy(cntv, cnt_hbm.at[wid])

    return part_kernel(src_all, dst_all)


@jax.jit
def _segmax(m, counts, srcl, dstl):
    @functools.partial(
        pl.kernel,
        mesh=_mesh,
        out_type=jax.ShapeDtypeStruct((NPAD, DD), jnp.float32),
        scratch_types=[
            pltpu.VMEM((16,), jnp.int32),
            pltpu.VMEM((GW,), jnp.int32),
            pltpu.VMEM((GW,), jnp.int32),
            pltpu.VMEM((GW, DD), jnp.float32),
            pltpu.VMEM((NPW + 1, DD), jnp.float32),
        ],
    )
    def seg_kernel(m_hbm, cnt_hbm, srcl_hbm, dstl_hbm, out_hbm,
                   cntv, idxv, dlv, rows, acc):
        wid = _wid()
        pltpu.sync_copy(cnt_hbm.at[wid], cntv)
        total = cntv[0]
        n_g = (total + GW - 1) // GW

        zero16 = jnp.zeros((16,), jnp.float32)

        @pl.loop(0, NPW + 1)
        def _(r):
            for j in range(DD // 16):
                acc[r, pl.ds(j * 16, 16)] = zero16

        def g_body(g, carry):
            pltpu.sync_copy(srcl_hbm.at[wid, pl.ds(g * GW, GW)], idxv)
            pltpu.sync_copy(dstl_hbm.at[wid, pl.ds(g * GW, GW)], dlv)
            pltpu.sync_copy(m_hbm.at[idxv], rows)

            def e_body(e, c2):
                dl = dlv[e]
                for j in range(DD // 16):
                    sl = pl.ds(j * 16, 16)
                    acc[dl, sl] = jnp.maximum(acc[dl, sl], rows[e, sl])
                return c2

            lax.fori_loop(0, GW, e_body, 0)
            return carry

        lax.fori_loop(0, n_g, g_body, 0)
        pltpu.sync_copy(acc.at[pl.ds(0, NPW)],
                        out_hbm.at[pl.ds(wid * NPW, NPW)])

    return seg_kernel(m, counts, srcl, dstl)


_PREC = jax.lax.Precision.HIGHEST


def _mm(a, b):
    return jnp.dot(a, b, preferred_element_type=jnp.float32, precision=_PREC)


def _leaky(h):
    return jnp.where(h > 0, h, 0.01 * h)


def _d0_body(x_ref, wp_ref, bp_ref, ws_ref, m_ref, xs_ref):
    x = x_ref[...]
    m_ref[...] = jnp.maximum(_mm(x, wp_ref[...]) + bp_ref[...], 0.0)
    xs_ref[...] = _mm(x, ws_ref[...])


@jax.jit
def _dense0(x, wp, bp, ws):
    return pl.pallas_call(
        _d0_body,
        out_shape=[jax.ShapeDtypeStruct((NN, DD), jnp.float32),
                   jax.ShapeDtypeStruct((NN, DD), jnp.float32)],
    )(x, wp, bp, ws)


def _mid_body(xs_ref, pooled_ref, wn_ref, b_ref, wp_ref, bp_ref, ws_ref,
              m_ref, xs2_ref):
    h = xs_ref[...] + _mm(pooled_ref[...], wn_ref[...]) + b_ref[...]
    h = _leaky(h)
    m_ref[...] = jnp.maximum(_mm(h, wp_ref[...]) + bp_ref[...], 0.0)
    xs2_ref[...] = _mm(h, ws_ref[...])


@jax.jit
def _dense_mid(xs, pooled, wn, b, wp, bp, ws):
    return pl.pallas_call(
        _mid_body,
        out_shape=[jax.ShapeDtypeStruct((NN, DD), jnp.float32),
                   jax.ShapeDtypeStruct((NN, DD), jnp.float32)],
    )(xs, pooled, wn, b, wp, bp, ws)


def _bn_body(xs_ref, pooled_ref, wn_ref, b_ref, g_ref, be_ref,
             wp_ref, bp_ref, ws_ref, m_ref, xs2_ref):
    h = xs_ref[...] + _mm(pooled_ref[...], wn_ref[...]) + b_ref[...]
    mu = jnp.mean(h, axis=0, keepdims=True)
    var = jnp.mean(jnp.square(h - mu), axis=0, keepdims=True)
    h = (h - mu) * jax.lax.rsqrt(var + 1e-5) * g_ref[...] + be_ref[...]
    h = _leaky(h)
    m_ref[...] = jnp.maximum(_mm(h, wp_ref[...]) + bp_ref[...], 0.0)
    xs2_ref[...] = _mm(h, ws_ref[...])


@jax.jit
def _dense_bn(xs, pooled, wn, b, g, be, wp, bp, ws):
    return pl.pallas_call(
        _bn_body,
        out_shape=[jax.ShapeDtypeStruct((NN, DD), jnp.float32),
                   jax.ShapeDtypeStruct((NN, DD), jnp.float32)],
    )(xs, pooled, wn, b, g, be, wp, bp, ws)


def _fin_body(xs_ref, pooled_ref, wn_ref, b_ref, wc_ref, bc_ref, out_ref):
    h = xs_ref[...] + _mm(pooled_ref[...], wn_ref[...]) + b_ref[...]
    out_ref[...] = _mm(h, wc_ref[...]) + bc_ref[...]


@jax.jit
def _dense_fin(xs, pooled, wn, b, wc, bc):
    return pl.pallas_call(
        _fin_body,
        out_shape=jax.ShapeDtypeStruct((NN, 40), jnp.float32),
    )(xs, pooled, wn, b, wc, bc)


def kernel(x, edge_index, Wp0, bp0, Ws0, Wn0, b0, Wp1, bp1, Ws1, Wn1, b1,
           g1, be1, Wp2, bp2, Ws2, Wn2, b2, Wc, bc):
    src = edge_index[0]
    dst = edge_index[1]
    counts, srcl, dstl = _partition(src, dst)

    bp0r = bp0.reshape(1, DD); b0r = b0.reshape(1, DD)
    bp1r = bp1.reshape(1, DD); b1r = b1.reshape(1, DD)
    bp2r = bp2.reshape(1, DD); b2r = b2.reshape(1, DD)
    be1r = be1.reshape(1, DD)
    g1r = g1.reshape(1, DD)
    bcr = bc.reshape(1, 40)

    m0, xs0 = _dense0(x, Wp0, bp0r, Ws0)
    pooled0 = _segmax(m0, counts, srcl, dstl)[:NN]
    m1, xs1 = _dense_mid(xs0, pooled0, Wn0, b0r, Wp1, bp1r, Ws1)
    pooled1 = _segmax(m1, counts, srcl, dstl)[:NN]
    m2, xs2 = _dense_bn(xs1, pooled1, Wn1, b1r, g1r, be1r, Wp2, bp2r, Ws2)
    pooled2 = _segmax(m2, counts, srcl, dstl)[:NN]
    return _dense_fin(xs2, pooled2, Wn2, b2r, Wc, bcr)


# speedup vs baseline: 2.0327x; 2.0327x over previous
import dataclasses
import functools

import jax
import jax.numpy as jnp
from jax import lax
from jax.experimental import pallas as pl
from jax.experimental.pallas import tpu as pltpu
from jax.experimental.pallas import tpu_sc as plsc

NN = 10000
EE = 320000
DD = 128

NWORK = 32
NPW = 320
NPAD = NWORK * NPW
WIN = 2000
GW = 256
CAP = EE + 8 * WIN

_mesh = plsc.VectorSubcoreMesh(core_axis_name="c", subcore_axis_name="s")

_SC_PARAMS = pltpu.CompilerParams()
if "needs_layout_passes" in pltpu.CompilerParams.__dataclass_fields__:
    _SC_PARAMS = dataclasses.replace(_SC_PARAMS, needs_layout_passes=False)

_TC_PARAMS = pltpu.CompilerParams(vmem_limit_bytes=100 * 1024 * 1024)


def _wid():
    return lax.axis_index("s") * 2 + lax.axis_index("c")


@jax.jit
def _partition(src_all, dst_all):
    @functools.partial(
        pl.kernel,
        mesh=_mesh,
        compiler_params=_SC_PARAMS,
        out_type=[
            jax.ShapeDtypeStruct((NWORK * 16,), jnp.int32),
            jax.ShapeDtypeStruct((NWORK * CAP,), jnp.int32),
            jax.ShapeDtypeStruct((NWORK * CAP,), jnp.int32),
        ],
        scratch_types=[
            pltpu.VMEM((WIN,), jnp.int32),
            pltpu.VMEM((WIN,), jnp.int32),
            pltpu.VMEM((WIN,), jnp.int32),
            pltpu.VMEM((WIN,), jnp.int32),
            pltpu.VMEM((16,), jnp.int32),
        ],
    )
    def part_kernel(src_hbm, dst_hbm, cnt_hbm, srcl_hbm, dstl_hbm,
                    srcw, dstw, stg_s, stg_d, cntv):
        wid = _wid()
        lo = wid * NPW
        hi = lo + NPW

        sent_s = jnp.full((16,), 0, jnp.int32) + lo
        sent_d = jnp.full((16,), NPW, jnp.int32)

        @pl.loop(0, WIN // 16)
        def _(i):
            stg_s[pl.ds(i * 16, 16)] = sent_s
            stg_d[pl.ds(i * 16, 16)] = sent_d

        def win_body(g, total):
            pltpu.sync_copy(src_hbm.at[pl.ds(g * WIN, WIN)], srcw)
            pltpu.sync_copy(dst_hbm.at[pl.ds(g * WIN, WIN)], dstw)

            def chunk(i, cnt):
                d = dstw[pl.ds(i * 16, 16)]
                s = srcw[pl.ds(i * 16, 16)]
                msk = (d >= lo) & (d < hi)
                pos = plsc.cumsum(msk.astype(jnp.int32)) + (cnt - 1)
                plsc.store_scatter(stg_s, [pos], s, mask=msk)
                plsc.store_scatter(stg_d, [pos], d - lo, mask=msk)
                pc = plsc.all_reduce_population_count(msk)
                return cnt + pc[0]

            cnt = lax.fori_loop(0, WIN // 16, chunk, 0)
            off = pl.multiple_of(wid * CAP + total, 16)
            pltpu.sync_copy(stg_s, srcl_hbm.at[pl.ds(off, WIN)])
            pltpu.sync_copy(stg_d, dstl_hbm.at[pl.ds(off, WIN)])
            return total + ((cnt + 15) & ~15)

        total = lax.fori_loop(0, EE // WIN, win_body, 0)
        off = pl.multiple_of(wid * CAP + total, 16)
        pltpu.sync_copy(stg_s, srcl_hbm.at[pl.ds(off, WIN)])
        pltpu.sync_copy(stg_d, dstl_hbm.at[pl.ds(off, WIN)])
        cntv[...] = jnp.full((16,), 0, jnp.int32) + total
        pltpu.sync_copy(cntv, cnt_hbm.at[pl.ds(pl.multiple_of(wid * 16, 16), 16)])

    return part_kernel(src_all, dst_all)


@jax.jit
def _segmax(m, counts, srcl, dstl):
    @functools.partial(
        pl.kernel,
        mesh=_mesh,
        out_type=jax.ShapeDtypeStruct((NPAD, DD), jnp.float32),
        scratch_types=[
            pltpu.VMEM((16,), jnp.int32),
            pltpu.VMEM((GW,), jnp.int32),
            pltpu.VMEM((GW,), jnp.int32),
            pltpu.VMEM((GW, DD), jnp.float32),
            pltpu.VMEM((NPW + 1, DD), jnp.float32),
        ],
    )
    def seg_kernel(m_hbm, cnt_hbm, srcl_hbm, dstl_hbm, out_hbm,
                   cntv, idxv, dlv, rows, acc):
        wid = _wid()
        pltpu.sync_copy(cnt_hbm.at[pl.ds(pl.multiple_of(wid * 16, 16), 16)], cntv)
        total = cntv[...][0]
        n_g = (total + GW - 1) // GW

        zero16 = jnp.zeros((16,), jnp.float32)

        @pl.loop(0, NPW + 1)
        def _(r):
            for j in range(DD // 16):
                acc[r, pl.ds(j * 16, 16)] = zero16

        def g_body(g, carry):
            goff = pl.multiple_of(wid * CAP + g * GW, 16)
            pltpu.sync_copy(srcl_hbm.at[pl.ds(goff, GW)], idxv)
            pltpu.sync_copy(dstl_hbm.at[pl.ds(goff, GW)], dlv)
            pltpu.sync_copy(m_hbm.at[idxv], rows)

            def e_body(c, c2):
                dl16 = dlv[pl.ds(c * 16, 16)]
                for k in range(16):
                    dl = dl16[k]
                    for j in range(DD // 16):
                        sl = pl.ds(j * 16, 16)
                        acc[dl, sl] = jnp.maximum(acc[dl, sl],
                                                  rows[c * 16 + k, sl])
                return c2

            lax.fori_loop(0, GW // 16, e_body, 0)
            return carry

        lax.fori_loop(0, n_g, g_body, 0)
        pltpu.sync_copy(acc.at[pl.ds(0, NPW)],
                        out_hbm.at[pl.ds(wid * NPW, NPW)])

    return seg_kernel(m, counts, srcl, dstl)


_PREC = jax.lax.Precision.HIGHEST


def _mm(a, b):
    return jnp.dot(a, b, preferred_element_type=jnp.float32, precision=_PREC)


def _leaky(h):
    return jnp.where(h > 0, h, 0.01 * h)


def _d0_body(x_ref, wp_ref, bp_ref, ws_ref, m_ref, xs_ref):
    x = x_ref[...]
    m_ref[...] = jnp.maximum(_mm(x, wp_ref[...]) + bp_ref[...], 0.0)
    xs_ref[...] = _mm(x, ws_ref[...])


@jax.jit
def _dense0(x, wp, bp, ws):
    return pl.pallas_call(
        _d0_body,
        out_shape=[jax.ShapeDtypeStruct((NN, DD), jnp.float32),
                   jax.ShapeDtypeStruct((NN, DD), jnp.float32)],
        compiler_params=_TC_PARAMS,
    )(x, wp, bp, ws)


def _mid_body(xs_ref, pooled_ref, wn_ref, b_ref, wp_ref, bp_ref, ws_ref,
              m_ref, xs2_ref):
    h = xs_ref[...] + _mm(pooled_ref[...], wn_ref[...]) + b_ref[...]
    h = _leaky(h)
    m_ref[...] = jnp.maximum(_mm(h, wp_ref[...]) + bp_ref[...], 0.0)
    xs2_ref[...] = _mm(h, ws_ref[...])


@jax.jit
def _dense_mid(xs, pooled, wn, b, wp, bp, ws):
    return pl.pallas_call(
        _mid_body,
        out_shape=[jax.ShapeDtypeStruct((NN, DD), jnp.float32),
                   jax.ShapeDtypeStruct((NN, DD), jnp.float32)],
        compiler_params=_TC_PARAMS,
    )(xs, pooled, wn, b, wp, bp, ws)


def _bn_body(xs_ref, pooled_ref, wn_ref, b_ref, g_ref, be_ref,
             wp_ref, bp_ref, ws_ref, m_ref, xs2_ref):
    h = xs_ref[...] + _mm(pooled_ref[...], wn_ref[...]) + b_ref[...]
    mu = jnp.mean(h, axis=0, keepdims=True)
    var = jnp.mean(jnp.square(h - mu), axis=0, keepdims=True)
    h = (h - mu) * jax.lax.rsqrt(var + 1e-5) * g_ref[...] + be_ref[...]
    h = _leaky(h)
    m_ref[...] = jnp.maximum(_mm(h, wp_ref[...]) + bp_ref[...], 0.0)
    xs2_ref[...] = _mm(h, ws_ref[...])


@jax.jit
def _dense_bn(xs, pooled, wn, b, g, be, wp, bp, ws):
    return pl.pallas_call(
        _bn_body,
        out_shape=[jax.ShapeDtypeStruct((NN, DD), jnp.float32),
                   jax.ShapeDtypeStruct((NN, DD), jnp.float32)],
        compiler_params=_TC_PARAMS,
    )(xs, pooled, wn, b, g, be, wp, bp, ws)


def _fin_body(xs_ref, pooled_ref, wn_ref, b_ref, wc_ref, bc_ref, out_ref):
    h = xs_ref[...] + _mm(pooled_ref[...], wn_ref[...]) + b_ref[...]
    out_ref[...] = _mm(h, wc_ref[...]) + bc_ref[...]


@jax.jit
def _dense_fin(xs, pooled, wn, b, wc, bc):
    return pl.pallas_call(
        _fin_body,
        out_shape=jax.ShapeDtypeStruct((NN, 40), jnp.float32),
        compiler_params=_TC_PARAMS,
    )(xs, pooled, wn, b, wc, bc)


def kernel(x, edge_index, Wp0, bp0, Ws0, Wn0, b0, Wp1, bp1, Ws1, Wn1, b1,
           g1, be1, Wp2, bp2, Ws2, Wn2, b2, Wc, bc):
    src = edge_index[0]
    dst = edge_index[1]
    counts, srcl, dstl = _partition(src, dst)

    bp0r = bp0.reshape(1, DD); b0r = b0.reshape(1, DD)
    bp1r = bp1.reshape(1, DD); b1r = b1.reshape(1, DD)
    bp2r = bp2.reshape(1, DD); b2r = b2.reshape(1, DD)
    be1r = be1.reshape(1, DD)
    g1r = g1.reshape(1, DD)
    bcr = bc.reshape(1, 40)

    m0, xs0 = _dense0(x, Wp0, bp0r, Ws0)
    pooled0 = _segmax(m0, counts, srcl, dstl)[:NN]
    m1, xs1 = _dense_mid(xs0, pooled0, Wn0, b0r, Wp1, bp1r, Ws1)
    pooled1 = _segmax(m1, counts, srcl, dstl)[:NN]
    m2, xs2 = _dense_bn(xs1, pooled1, Wn1, b1r, g1r, be1r, Wp2, bp2r, Ws2)
    pooled2 = _segmax(m2, counts, srcl, dstl)[:NN]
    return _dense_fin(xs2, pooled2, Wn2, b2r, Wc, bcr)


# speedup vs baseline: 2.3482x; 1.1552x over previous
import dataclasses
import functools

import jax
import jax.numpy as jnp
from jax import lax
from jax.experimental import pallas as pl
from jax.experimental.pallas import tpu as pltpu
from jax.experimental.pallas import tpu_sc as plsc

NN = 10000
EE = 320000
DD = 128

NWORK = 32
NPW = 320
NPAD = NWORK * NPW
WIN = 2000
GW = 256
CAP = EE + 8 * WIN

_mesh = plsc.VectorSubcoreMesh(core_axis_name="c", subcore_axis_name="s")

_SC_PARAMS = pltpu.CompilerParams()
if "needs_layout_passes" in pltpu.CompilerParams.__dataclass_fields__:
    _SC_PARAMS = dataclasses.replace(_SC_PARAMS, needs_layout_passes=False)

_TC_PARAMS = pltpu.CompilerParams(vmem_limit_bytes=100 * 1024 * 1024)


def _wid():
    return lax.axis_index("s") * 2 + lax.axis_index("c")


@jax.jit
def _partition(src_all, dst_all):
    @functools.partial(
        pl.kernel,
        mesh=_mesh,
        compiler_params=_SC_PARAMS,
        out_type=[
            jax.ShapeDtypeStruct((NWORK * 16,), jnp.int32),
            jax.ShapeDtypeStruct((NWORK * CAP,), jnp.int32),
            jax.ShapeDtypeStruct((NWORK * CAP,), jnp.int32),
        ],
        scratch_types=[
            pltpu.VMEM((WIN,), jnp.int32),
            pltpu.VMEM((WIN,), jnp.int32),
            pltpu.VMEM((WIN,), jnp.int32),
            pltpu.VMEM((WIN,), jnp.int32),
            pltpu.VMEM((16,), jnp.int32),
        ],
    )
    def part_kernel(src_hbm, dst_hbm, cnt_hbm, srcl_hbm, dstl_hbm,
                    srcw, dstw, stg_s, stg_d, cntv):
        wid = _wid()
        lo = wid * NPW
        hi = lo + NPW

        sent_s = jnp.full((16,), 0, jnp.int32) + lo
        sent_d = jnp.full((16,), NPW, jnp.int32)

        @pl.loop(0, WIN // 16)
        def _(i):
            stg_s[pl.ds(i * 16, 16)] = sent_s
            stg_d[pl.ds(i * 16, 16)] = sent_d

        def win_body(g, total):
            pltpu.sync_copy(src_hbm.at[pl.ds(g * WIN, WIN)], srcw)
            pltpu.sync_copy(dst_hbm.at[pl.ds(g * WIN, WIN)], dstw)

            def chunk(i, cntv_):
                d = dstw[pl.ds(i * 16, 16)]
                s = srcw[pl.ds(i * 16, 16)]
                msk = (d >= lo) & (d < hi)
                pos = plsc.cumsum(msk.astype(jnp.int32)) + (cntv_ - 1)
                plsc.store_scatter(stg_s, [pos], s, mask=msk)
                plsc.store_scatter(stg_d, [pos], d - lo, mask=msk)
                pc = plsc.all_reduce_population_count(msk)
                return cntv_ + pc

            cnt_vec = lax.fori_loop(0, WIN // 16, chunk,
                                    jnp.zeros((16,), jnp.int32), unroll=4)
            cnt = cnt_vec[0]
            off = pl.multiple_of(wid * CAP + total, 16)
            pltpu.sync_copy(stg_s, srcl_hbm.at[pl.ds(off, WIN)])
            pltpu.sync_copy(stg_d, dstl_hbm.at[pl.ds(off, WIN)])
            return total + ((cnt + 15) & ~15)

        total = lax.fori_loop(0, EE // WIN, win_body, 0)
        off = pl.multiple_of(wid * CAP + total, 16)
        pltpu.sync_copy(stg_s, srcl_hbm.at[pl.ds(off, WIN)])
        pltpu.sync_copy(stg_d, dstl_hbm.at[pl.ds(off, WIN)])
        cntv[...] = jnp.full((16,), 0, jnp.int32) + total
        pltpu.sync_copy(cntv, cnt_hbm.at[pl.ds(pl.multiple_of(wid * 16, 16), 16)])

    return part_kernel(src_all, dst_all)


@jax.jit
def _segmax(m, counts, srcl, dstl):
    @functools.partial(
        pl.kernel,
        mesh=_mesh,
        out_type=jax.ShapeDtypeStruct((NPAD, DD), jnp.float32),
        scratch_types=[
            pltpu.VMEM((16,), jnp.int32),
            pltpu.VMEM((GW,), jnp.int32),
            pltpu.VMEM((GW,), jnp.int32),
            pltpu.VMEM((GW,), jnp.int32),
            pltpu.VMEM((GW,), jnp.int32),
            pltpu.VMEM((GW, DD), jnp.float32),
            pltpu.VMEM((GW, DD), jnp.float32),
            pltpu.VMEM((NPW + 1, DD), jnp.float32),
            pltpu.SemaphoreType.DMA,
            pltpu.SemaphoreType.DMA,
        ],
    )
    def seg_kernel(m_hbm, cnt_hbm, srcl_hbm, dstl_hbm, out_hbm,
                   cntv, idx0, idx1, dl0, dl1, rows0, rows1, acc,
                   sem0, sem1):
        wid = _wid()
        pltpu.sync_copy(cnt_hbm.at[pl.ds(pl.multiple_of(wid * 16, 16), 16)], cntv)
        total = cntv[...][0]
        n_g = (total + GW - 1) // GW

        idxs = (idx0, idx1)
        dls = (dl0, dl1)
        rowss = (rows0, rows1)
        sems = (sem0, sem1)

        zero16 = jnp.zeros((16,), jnp.float32)

        @pl.loop(0, NPW + 1)
        def _(r):
            for j in range(DD // 16):
                acc[r, pl.ds(j * 16, 16)] = zero16

        def issue(g, slot):
            goff = pl.multiple_of(wid * CAP + g * GW, 16)
            pltpu.sync_copy(srcl_hbm.at[pl.ds(goff, GW)], idxs[slot])
            pltpu.sync_copy(dstl_hbm.at[pl.ds(goff, GW)], dls[slot])
            pltpu.async_copy(m_hbm.at[idxs[slot]], rowss[slot], sems[slot])

        def accum(slot):
            rows = rowss[slot]
            dlv = dls[slot]
            pltpu.make_async_copy(m_hbm.at[idxs[slot]], rows,
                                  sems[slot]).wait()

            def e_body(c, c2):
                dl16 = dlv[pl.ds(c * 16, 16)]
                for k in range(16):
                    dl = dl16[k]
                    for j in range(DD // 16):
                        sl = pl.ds(j * 16, 16)
                        acc[dl, sl] = jnp.maximum(acc[dl, sl],
                                                  rows[c * 16 + k, sl])
                return c2

            lax.fori_loop(0, GW // 16, e_body, 0)

        @pl.when(n_g > 0)
        def _():
            issue(0, 0)

        def g_body(g, carry):
            def stage(slot, other):
                @pl.when(g + 1 < n_g)
                def _():
                    issue(g + 1, other)

                accum(slot)

            @pl.when(lax.rem(g, 2) == 0)
            def _():
                stage(0, 1)

            @pl.when(lax.rem(g, 2) == 1)
            def _():
                stage(1, 0)

            return carry

        lax.fori_loop(0, n_g, g_body, 0)
        pltpu.sync_copy(acc.at[pl.ds(0, NPW)],
                        out_hbm.at[pl.ds(wid * NPW, NPW)])

    return seg_kernel(m, counts, srcl, dstl)


_PREC = jax.lax.Precision.HIGHEST


def _mm(a, b):
    return jnp.dot(a, b, preferred_element_type=jnp.float32, precision=_PREC)


def _leaky(h):
    return jnp.where(h > 0, h, 0.01 * h)


def _d0_body(x_ref, wp_ref, bp_ref, ws_ref, m_ref, xs_ref):
    x = x_ref[...]
    m_ref[...] = jnp.maximum(_mm(x, wp_ref[...]) + bp_ref[...], 0.0)
    xs_ref[...] = _mm(x, ws_ref[...])


@jax.jit
def _dense0(x, wp, bp, ws):
    return pl.pallas_call(
        _d0_body,
        out_shape=[jax.ShapeDtypeStruct((NN, DD), jnp.float32),
                   jax.ShapeDtypeStruct((NN, DD), jnp.float32)],
        compiler_params=_TC_PARAMS,
    )(x, wp, bp, ws)


def _mid_body(xs_ref, pooled_ref, wn_ref, b_ref, wp_ref, bp_ref, ws_ref,
              m_ref, xs2_ref):
    h = xs_ref[...] + _mm(pooled_ref[...], wn_ref[...]) + b_ref[...]
    h = _leaky(h)
    m_ref[...] = jnp.maximum(_mm(h, wp_ref[...]) + bp_ref[...], 0.0)
    xs2_ref[...] = _mm(h, ws_ref[...])


@jax.jit
def _dense_mid(xs, pooled, wn, b, wp, bp, ws):
    return pl.pallas_call(
        _mid_body,
        out_shape=[jax.ShapeDtypeStruct((NN, DD), jnp.float32),
                   jax.ShapeDtypeStruct((NN, DD), jnp.float32)],
        compiler_params=_TC_PARAMS,
    )(xs, pooled, wn, b, wp, bp, ws)


def _bn_body(xs_ref, pooled_ref, wn_ref, b_ref, g_ref, be_ref,
             wp_ref, bp_ref, ws_ref, m_ref, xs2_ref):
    h = xs_ref[...] + _mm(pooled_ref[...], wn_ref[...]) + b_ref[...]
    mu = jnp.mean(h, axis=0, keepdims=True)
    var = jnp.mean(jnp.square(h - mu), axis=0, keepdims=True)
    h = (h - mu) * jax.lax.rsqrt(var + 1e-5) * g_ref[...] + be_ref[...]
    h = _leaky(h)
    m_ref[...] = jnp.maximum(_mm(h, wp_ref[...]) + bp_ref[...], 0.0)
    xs2_ref[...] = _mm(h, ws_ref[...])


@jax.jit
def _dense_bn(xs, pooled, wn, b, g, be, wp, bp, ws):
    return pl.pallas_call(
        _bn_body,
        out_shape=[jax.ShapeDtypeStruct((NN, DD), jnp.float32),
                   jax.ShapeDtypeStruct((NN, DD), jnp.float32)],
        compiler_params=_TC_PARAMS,
    )(xs, pooled, wn, b, g, be, wp, bp, ws)


def _fin_body(xs_ref, pooled_ref, wn_ref, b_ref, wc_ref, bc_ref, out_ref):
    h = xs_ref[...] + _mm(pooled_ref[...], wn_ref[...]) + b_ref[...]
    out_ref[...] = _mm(h, wc_ref[...]) + bc_ref[...]


@jax.jit
def _dense_fin(xs, pooled, wn, b, wc, bc):
    return pl.pallas_call(
        _fin_body,
        out_shape=jax.ShapeDtypeStruct((NN, 40), jnp.float32),
        compiler_params=_TC_PARAMS,
    )(xs, pooled, wn, b, wc, bc)


def kernel(x, edge_index, Wp0, bp0, Ws0, Wn0, b0, Wp1, bp1, Ws1, Wn1, b1,
           g1, be1, Wp2, bp2, Ws2, Wn2, b2, Wc, bc):
    src = edge_index[0]
    dst = edge_index[1]
    counts, srcl, dstl = _partition(src, dst)

    bp0r = bp0.reshape(1, DD); b0r = b0.reshape(1, DD)
    bp1r = bp1.reshape(1, DD); b1r = b1.reshape(1, DD)
    bp2r = bp2.reshape(1, DD); b2r = b2.reshape(1, DD)
    be1r = be1.reshape(1, DD)
    g1r = g1.reshape(1, DD)
    bcr = bc.reshape(1, 40)

    m0, xs0 = _dense0(x, Wp0, bp0r, Ws0)
    pooled0 = _segmax(m0, counts, srcl, dstl)[:NN]
    m1, xs1 = _dense_mid(xs0, pooled0, Wn0, b0r, Wp1, bp1r, Ws1)
    pooled1 = _segmax(m1, counts, srcl, dstl)[:NN]
    m2, xs2 = _dense_bn(xs1, pooled1, Wn1, b1r, g1r, be1r, Wp2, bp2r, Ws2)
    pooled2 = _segmax(m2, counts, srcl, dstl)[:NN]
    return _dense_fin(xs2, pooled2, Wn2, b2r, Wc, bcr)
